# baseline (speedup 1.0000x reference)
import functools

import jax
import jax.numpy as jnp
from jax import lax
from jax.experimental import pallas as pl
from jax.experimental.pallas import tpu as pltpu
from jax.experimental.pallas import tpu_sc as plsc

DIM = 1024
E = 8
K = 2
FFN = 2048
N = 2048
NK = N * K
EPS = 1e-5

B = 256
NB = 24
M = NB * B

NC = 2
NS = 16
NW = NC * NS
CH = NK // NS
RW = M // NW
GB = 32
QW = NK // NW

_INV_SQRT2 = 0.7071067811865476


def _gelu_exact(h):
    return 0.5 * h * (1.0 + jax.lax.erf(h * _INV_SQRT2))


def _gate_body(x_ref, wg_ref, ti_ref, w_ref):
    logits = jnp.dot(x_ref[...], wg_ref[...], preferred_element_type=jnp.float32)
    cols = jax.lax.broadcasted_iota(jnp.int32, (N, E), 1)
    m1 = jnp.max(logits, axis=1, keepdims=True)
    i1 = jnp.min(jnp.where(logits == m1, cols, E), axis=1, keepdims=True)
    neg = jnp.float32(-jnp.inf)
    logits2 = jnp.where(cols == i1, neg, logits)
    m2 = jnp.max(logits2, axis=1, keepdims=True)
    i2 = jnp.min(jnp.where(logits2 == m2, cols, E), axis=1, keepdims=True)
    s = jnp.exp(m2 - m1)
    w0 = 1.0 / (1.0 + s)
    w1 = s / (1.0 + s)
    ti_ref[...] = jnp.concatenate([i1, i2], axis=1)
    w_ref[...] = jnp.concatenate([w0, w1], axis=1)


def _gate(x, Wg):
    return pl.pallas_call(
        _gate_body,
        out_shape=(jax.ShapeDtypeStruct((N, K), jnp.int32),
                   jax.ShapeDtypeStruct((N, K), jnp.float32)),
    )(x, Wg)


def _route_body(tflat_hbm, x_hbm, xs_hbm, posj_hbm, bmeta_hbm,
                idsv, tmpv, destv, jv, histfv, zb, gidxv, sjv, gbuf,
                cnt_s, start_s, csum_s,
                hist_sh, sj_sh, sem):
    cid = lax.axis_index("c")
    sid = lax.axis_index("s")
    wid = sid * NC + cid
    lanes = lax.broadcasted_iota(jnp.int32, (16,), 0)

    pltpu.sync_copy(tflat_hbm.at[pl.ds(sid * CH, CH)], idsv)
    for l in range(16):
        cnt_s[l] = jnp.int32(0)
    for v in range(CH // 16):
        vec = idsv[pl.ds(v * 16, 16)]
        for l in range(16):
            e = vec[l]
            cnt_s[e] = cnt_s[e] + 1
    hv = jnp.zeros((16,), jnp.int32)
    for l in range(16):
        hv = jnp.where(lanes == l, cnt_s[l], hv)
    tmpv[...] = hv
    pltpu.sync_copy(tmpv, hist_sh.at[pl.ds(sid * 16, 16)])

    zb[...] = jnp.zeros((16,), jnp.int32)
    for z in range(RW * NC // 16):
        pltpu.sync_copy(zb, sj_sh.at[pl.ds(sid * (RW * NC) + z * 16, 16)])
    plsc.subcore_barrier()

    pltpu.sync_copy(hist_sh, histfv)
    tot = jnp.zeros((16,), jnp.int32)
    mypre = jnp.zeros((16,), jnp.int32)
    for s in range(NS):
        row = histfv[pl.ds(s * 16, 16)]
        tot = tot + row
        mypre = mypre + jnp.where(s < sid, row, 0)
    padded = ((tot + (B - 1)) >> 8) << 8
    run = jnp.int32(0)
    for e in range(E):
        run = run + padded[e]
        csum_s[e] = run
        start_s[e] = run - padded[e] + mypre[e]

    @pl.when(jnp.logical_and(cid == 0, sid == 0))
    def _bmeta():
        ones = jnp.ones((16,), jnp.int32)
        zeros = jnp.zeros((16,), jnp.int32)
        for half in range(2):
            boff = (lanes + half * 16) * B
            acc = jnp.zeros((16,), jnp.int32)
            for e in range(E - 1):
                cev = jnp.where(lanes >= 0, csum_s[e], 0)
                acc = jnp.where(boff >= cev, acc + ones, acc)
            tmpv[...] = jnp.minimum(acc, E - 1)
            pltpu.sync_copy(tmpv, bmeta_hbm.at[0, pl.ds(half * 16, 16)])
            ctv = jnp.where(lanes >= 0, csum_s[E - 1], 0)
            tmpv[...] = jnp.where(boff < ctv, ones, zeros)
            pltpu.sync_copy(tmpv, bmeta_hbm.at[1, pl.ds(half * 16, 16)])

    for v in range(CH // 16):
        vec = idsv[pl.ds(v * 16, 16)]
        dvec = jnp.zeros((16,), jnp.int32)
        for l in range(16):
            e = vec[l]
            p = start_s[e]
            start_s[e] = p + 1
            dvec = jnp.where(lanes == l, p, dvec)
        destv[pl.ds(v * 16, 16)] = dvec
        jv[...] = sid * CH + v * 16 + lanes + 1
        pltpu.sync_copy(jv, sj_sh.at[dvec], add=True)
    pltpu.sync_copy(destv, posj_hbm.at[pl.ds(sid * CH, CH)])
    plsc.subcore_barrier()

    gbase = wid * RW
    pltpu.sync_copy(sj_sh.at[pl.ds(gbase, RW)], sjv)
    for v in range(RW // 16):
        s16 = sjv[pl.ds(v * 16, 16)]
        tok = jnp.where(s16 > 0, (s16 - 1) >> 1, 0)
        gidxv[pl.ds(v * 16, 16)] = tok
    for c in range(RW // GB):
        pltpu.async_copy(x_hbm.at[gidxv.at[pl.ds(c * GB, GB)]], gbuf, sem).wait()
        pltpu.sync_copy(gbuf, xs_hbm.at[pl.ds(gbase + c * GB, GB)])


def _route(tflat, x):
    mesh = plsc.VectorSubcoreMesh(core_axis_name="c", subcore_axis_name="s")
    kern = functools.partial(
        pl.kernel,
        mesh=mesh,
        out_type=(jax.ShapeDtypeStruct((M, DIM), jnp.float32),
                  jax.ShapeDtypeStruct((NK,), jnp.int32),
                  jax.ShapeDtypeStruct((2, 32), jnp.int32)),
        scratch_types=[
            pltpu.VMEM((CH,), jnp.int32),
            pltpu.VMEM((16,), jnp.int32),
            pltpu.VMEM((CH,), jnp.int32),
            pltpu.VMEM((16,), jnp.int32),
            pltpu.VMEM((NS * 16,), jnp.int32),
            pltpu.VMEM((16,), jnp.int32),
            pltpu.VMEM((RW,), jnp.int32),
            pltpu.VMEM((RW,), jnp.int32),
            pltpu.VMEM((GB, DIM), jnp.float32),
            pltpu.SMEM((16,), jnp.int32),
            pltpu.SMEM((16,), jnp.int32),
            pltpu.SMEM((16,), jnp.int32),
            pltpu.VMEM_SHARED((NS * 16,), jnp.int32),
            pltpu.VMEM_SHARED((M,), jnp.int32),
            pltpu.SemaphoreType.DMA,
        ],
    )(_route_body)
    return kern(tflat, x)


def _expert_body(s_ref, xs_ref, w1_ref, b1_ref, w2_ref, b2_ref, ys_ref):
    b = pl.program_id(0)

    @pl.when(s_ref[1, b] > 0)
    def _():
        xb = xs_ref[...].astype(jnp.bfloat16)
        h = jnp.dot(xb, w1_ref[0], preferred_element_type=jnp.float32)
        h = _gelu_exact(h + b1_ref[0])
        hb = h.astype(jnp.bfloat16)
        ys_ref[...] = (jnp.dot(hb, w2_ref[0], preferred_element_type=jnp.float32)
                       + b2_ref[0])


def _experts(bmeta, xs, W1, b1, W2, b2):
    grid_spec = pltpu.PrefetchScalarGridSpec(
        num_scalar_prefetch=1,
        grid=(NB,),
        in_specs=[
            pl.BlockSpec((B, DIM), lambda b, s: (b, 0)),
            pl.BlockSpec((1, DIM, FFN), lambda b, s: (s[0, b], 0, 0)),
            pl.BlockSpec((1, 1, FFN), lambda b, s: (s[0, b], 0, 0)),
            pl.BlockSpec((1, FFN, DIM), lambda b, s: (s[0, b], 0, 0)),
            pl.BlockSpec((1, 1, DIM), lambda b, s: (s[0, b], 0, 0)),
        ],
        out_specs=pl.BlockSpec((B, DIM), lambda b, s: (b, 0)),
    )
    return pl.pallas_call(
        _expert_body,
        grid_spec=grid_spec,
        out_shape=jax.ShapeDtypeStruct((M, DIM), jnp.float32),
    )(bmeta, xs, W1.astype(jnp.bfloat16), b1[:, None, :],
      W2.astype(jnp.bfloat16), b2[:, None, :])


def _combine_body(ys_hbm, posj_hbm, z_hbm, pidxv, gbuf, sem):
    cid = lax.axis_index("c")
    sid = lax.axis_index("s")
    wid = sid * NC + cid
    qbase = wid * QW
    pltpu.sync_copy(posj_hbm.at[pl.ds(qbase, QW)], pidxv)
    for c in range(QW // GB):
        pltpu.async_copy(ys_hbm.at[pidxv.at[pl.ds(c * GB, GB)]], gbuf, sem).wait()
        pltpu.sync_copy(gbuf, z_hbm.at[pl.ds(qbase + c * GB, GB)])


def _combine(ys, posj):
    mesh = plsc.VectorSubcoreMesh(core_axis_name="c", subcore_axis_name="s")
    kern = functools.partial(
        pl.kernel,
        mesh=mesh,
        out_type=jax.ShapeDtypeStruct((NK, DIM), jnp.float32),
        scratch_types=[
            pltpu.VMEM((QW,), jnp.int32),
            pltpu.VMEM((GB, DIM), jnp.float32),
            pltpu.SemaphoreType.DMA,
        ],
    )(_combine_body)
    return kern(ys, posj)


TBF = 256
TF = N // TBF


def _final_body(x_ref, w_ref, z_ref, ws1_ref, bs1_ref, ws2_ref, bs2_ref,
                gamma_ref, beta_ref, out_ref):
    x = x_ref[...]
    h = jnp.dot(x, ws1_ref[...], preferred_element_type=jnp.float32) + bs1_ref[...]
    h = _gelu_exact(h)
    sh = jnp.dot(h, ws2_ref[...], preferred_element_type=jnp.float32) + bs2_ref[...]
    z = z_ref[...]
    w = w_ref[...]
    acc = (x + sh + w[:, 0:1] * z[:, :DIM] + w[:, 1:2] * z[:, DIM:])
    mu = jnp.mean(acc, axis=1, keepdims=True)
    d = acc - mu
    var = jnp.mean(d * d, axis=1, keepdims=True)
    out_ref[...] = d * jax.lax.rsqrt(var + EPS) * gamma_ref[...] + beta_ref[...]


def _final(x, wts, z2, Ws1, bs1, Ws2, bs2, gamma, beta):
    return pl.pallas_call(
        _final_body,
        grid=(TF,),
        in_specs=[
            pl.BlockSpec((TBF, DIM), lambda t: (t, 0)),
            pl.BlockSpec((TBF, K), lambda t: (t, 0)),
            pl.BlockSpec((TBF, K * DIM), lambda t: (t, 0)),
            pl.BlockSpec((DIM, FFN), lambda t: (0, 0)),
            pl.BlockSpec((FFN,), lambda t: (0,)),
            pl.BlockSpec((FFN, DIM), lambda t: (0, 0)),
            pl.BlockSpec((DIM,), lambda t: (0,)),
            pl.BlockSpec((DIM,), lambda t: (0,)),
            pl.BlockSpec((DIM,), lambda t: (0,)),
        ],
        out_specs=pl.BlockSpec((TBF, DIM), lambda t: (t, 0)),
        out_shape=jax.ShapeDtypeStruct((N, DIM), jnp.float32),
    )(x, wts, z2, Ws1, bs1, Ws2, bs2, gamma, beta)


def kernel(x, Wg, W1, b1, W2, b2, Ws1, bs1, Ws2, bs2, gamma, beta):
    ti, wts = _gate(x, Wg)
    tflat = ti.reshape(NK)
    xs, posj, bmeta = _route(tflat, x)
    ys = _experts(bmeta, xs, W1, b1, W2, b2)
    z = _combine(ys, posj)
    z2 = z.reshape(N, K * DIM)
    return _final(x, wts, z2, Ws1, bs1, Ws2, bs2, gamma, beta)


# speedup vs baseline: 1.6148x; 1.6148x over previous
import functools

import jax
import jax.numpy as jnp
from jax import lax
from jax.experimental import pallas as pl
from jax.experimental.pallas import tpu as pltpu
from jax.experimental.pallas import tpu_sc as plsc

DIM = 1024
E = 8
K = 2
FFN = 2048
N = 2048
NK = N * K
EPS = 1e-5

B = 256
NB = 24
M = NB * B

NC = 2
NS = 16
NW = NC * NS
CH = NK // NS
RW = M // NW
GB = 32
QW = NK // NW

_INV_SQRT2 = 0.7071067811865476


def _gelu_exact(h):
    return 0.5 * h * (1.0 + jax.lax.erf(h * _INV_SQRT2))


def _gate_body(x_ref, wg_ref, ti_ref, w_ref):
    logits = jnp.dot(x_ref[...], wg_ref[...], preferred_element_type=jnp.float32)
    cols = jax.lax.broadcasted_iota(jnp.int32, (N, E), 1)
    m1 = jnp.max(logits, axis=1, keepdims=True)
    i1 = jnp.min(jnp.where(logits == m1, cols, E), axis=1, keepdims=True)
    neg = jnp.float32(-jnp.inf)
    logits2 = jnp.where(cols == i1, neg, logits)
    m2 = jnp.max(logits2, axis=1, keepdims=True)
    i2 = jnp.min(jnp.where(logits2 == m2, cols, E), axis=1, keepdims=True)
    s = jnp.exp(m2 - m1)
    w0 = 1.0 / (1.0 + s)
    w1 = s / (1.0 + s)
    ti_ref[...] = jnp.concatenate([i1, i2], axis=1)
    w_ref[...] = jnp.concatenate([w0, w1], axis=1)


def _gate(x, Wg):
    return pl.pallas_call(
        _gate_body,
        out_shape=(jax.ShapeDtypeStruct((N, K), jnp.int32),
                   jax.ShapeDtypeStruct((N, K), jnp.float32)),
    )(x, Wg)


def _route_body(tflat_hbm, x_hbm, xs_hbm, posj_hbm, bmeta_hbm,
                idsv, tmpv, dest0v, dest1v, histfv, gbuf,
                cnt_s, start_s, csum_s,
                hist_sh, sem):
    cid = lax.axis_index("c")
    sid = lax.axis_index("s")
    wid = sid * NC + cid
    lanes = lax.broadcasted_iota(jnp.int32, (16,), 0)

    pltpu.sync_copy(tflat_hbm.at[pl.ds(sid * CH, CH)], idsv)
    for half in range(2):
        for l in range(16):
            cnt_s[l] = jnp.int32(0)
        for v in range(8):
            vec = idsv[pl.ds(half * 128 + v * 16, 16)]
            for l in range(16):
                e = vec[l]
                cnt_s[e] = cnt_s[e] + 1
        hv = jnp.zeros((16,), jnp.int32)
        for l in range(16):
            hv = jnp.where(lanes == l, cnt_s[l], hv)
        tmpv[...] = hv
        pltpu.sync_copy(tmpv, hist_sh.at[pl.ds(sid * 32 + half * 16, 16)])
    plsc.subcore_barrier()

    pltpu.sync_copy(hist_sh, histfv)
    tot = jnp.zeros((16,), jnp.int32)
    mypre = jnp.zeros((16,), jnp.int32)
    for w in range(NW):
        row = histfv[pl.ds(w * 16, 16)]
        tot = tot + row
        mypre = mypre + jnp.where(w < wid, row, 0)
    padded = ((tot + (B - 1)) >> 8) << 8
    run = jnp.int32(0)
    for e in range(E):
        run = run + padded[e]
        csum_s[e] = run
        start_s[e] = run - padded[e] + mypre[e]

    @pl.when(jnp.logical_and(cid == 0, sid == 0))
    def _bmeta():
        ones = jnp.ones((16,), jnp.int32)
        zeros = jnp.zeros((16,), jnp.int32)
        for half in range(2):
            boff = (lanes + half * 16) * B
            acc = jnp.zeros((16,), jnp.int32)
            for e in range(E - 1):
                cev = jnp.where(lanes >= 0, csum_s[e], 0)
                acc = jnp.where(boff >= cev, acc + ones, acc)
            tmpv[...] = jnp.minimum(acc, E - 1)
            pltpu.sync_copy(tmpv, bmeta_hbm.at[0, pl.ds(half * 16, 16)])
            ctv = jnp.where(lanes >= 0, csum_s[E - 1], 0)
            tmpv[...] = jnp.where(boff < ctv, ones, zeros)
            pltpu.sync_copy(tmpv, bmeta_hbm.at[1, pl.ds(half * 16, 16)])

    base = cid * 128
    for tv in range(4):
        dvec0 = jnp.zeros((16,), jnp.int32)
        dvec1 = jnp.zeros((16,), jnp.int32)
        for half in range(2):
            vec = idsv[pl.ds(base + tv * 32 + half * 16, 16)]
            for l in range(16):
                e = vec[l]
                p = start_s[e]
                start_s[e] = p + 1
                lane = half * 8 + (l >> 1)
                if l % 2 == 0:
                    dvec0 = jnp.where(lanes == lane, p, dvec0)
                else:
                    dvec1 = jnp.where(lanes == lane, p, dvec1)
        dest0v[pl.ds(tv * 16, 16)] = dvec0
        dest1v[pl.ds(tv * 16, 16)] = dvec1
    tbase = wid * 64
    pltpu.sync_copy(dest0v, posj_hbm.at[0, pl.ds(tbase, 64)])
    pltpu.sync_copy(dest1v, posj_hbm.at[1, pl.ds(tbase, 64)])

    pltpu.sync_copy(x_hbm.at[pl.ds(tbase, 64)], gbuf)
    pltpu.sync_copy(gbuf, xs_hbm.at[dest0v])
    pltpu.sync_copy(gbuf, xs_hbm.at[dest1v])


def _route(tflat, x):
    mesh = plsc.VectorSubcoreMesh(core_axis_name="c", subcore_axis_name="s")
    kern = functools.partial(
        pl.kernel,
        mesh=mesh,
        out_type=(jax.ShapeDtypeStruct((M, DIM), jnp.float32),
                  jax.ShapeDtypeStruct((2, N), jnp.int32),
                  jax.ShapeDtypeStruct((2, 32), jnp.int32)),
        scratch_types=[
            pltpu.VMEM((CH,), jnp.int32),
            pltpu.VMEM((16,), jnp.int32),
            pltpu.VMEM((64,), jnp.int32),
            pltpu.VMEM((64,), jnp.int32),
            pltpu.VMEM((NW * 16,), jnp.int32),
            pltpu.VMEM((64, DIM), jnp.float32),
            pltpu.SMEM((16,), jnp.int32),
            pltpu.SMEM((16,), jnp.int32),
            pltpu.SMEM((16,), jnp.int32),
            pltpu.VMEM_SHARED((NW * 16,), jnp.int32),
            pltpu.SemaphoreType.DMA,
        ],
    )(_route_body)
    return kern(tflat, x)


def _expert_body(s_ref, xs_ref, w1_ref, b1_ref, w2_ref, b2_ref, ys_ref):
    b = pl.program_id(0)

    @pl.when(s_ref[1, b] > 0)
    def _():
        xb = xs_ref[...].astype(jnp.bfloat16)
        h = jnp.dot(xb, w1_ref[0], preferred_element_type=jnp.float32)
        h = _gelu_exact(h + b1_ref[0])
        hb = h.astype(jnp.bfloat16)
        ys_ref[...] = (jnp.dot(hb, w2_ref[0], preferred_element_type=jnp.float32)
                       + b2_ref[0])


def _experts(bmeta, xs, W1, b1, W2, b2):
    grid_spec = pltpu.PrefetchScalarGridSpec(
        num_scalar_prefetch=1,
        grid=(NB,),
        in_specs=[
            pl.BlockSpec((B, DIM), lambda b, s: (b, 0)),
            pl.BlockSpec((1, DIM, FFN), lambda b, s: (s[0, b], 0, 0)),
            pl.BlockSpec((1, 1, FFN), lambda b, s: (s[0, b], 0, 0)),
            pl.BlockSpec((1, FFN, DIM), lambda b, s: (s[0, b], 0, 0)),
            pl.BlockSpec((1, 1, DIM), lambda b, s: (s[0, b], 0, 0)),
        ],
        out_specs=pl.BlockSpec((B, DIM), lambda b, s: (b, 0)),
    )
    return pl.pallas_call(
        _expert_body,
        grid_spec=grid_spec,
        out_shape=jax.ShapeDtypeStruct((M, DIM), jnp.float32),
    )(bmeta, xs, W1.astype(jnp.bfloat16), b1[:, None, :],
      W2.astype(jnp.bfloat16), b2[:, None, :])


def _combine_body(ys_hbm, posj_hbm, z_hbm, p0v, p1v, gbuf, sem):
    cid = lax.axis_index("c")
    sid = lax.axis_index("s")
    wid = sid * NC + cid
    tbase = wid * 64
    pltpu.sync_copy(posj_hbm.at[0, pl.ds(tbase, 64)], p0v)
    pltpu.sync_copy(posj_hbm.at[1, pl.ds(tbase, 64)], p1v)
    pltpu.async_copy(ys_hbm.at[p0v], gbuf, sem).wait()
    pltpu.sync_copy(gbuf, z_hbm.at[pl.ds(tbase, 64)])
    pltpu.async_copy(ys_hbm.at[p1v], gbuf, sem).wait()
    pltpu.sync_copy(gbuf, z_hbm.at[pl.ds(N + tbase, 64)])


def _combine(ys, posj):
    mesh = plsc.VectorSubcoreMesh(core_axis_name="c", subcore_axis_name="s")
    kern = functools.partial(
        pl.kernel,
        mesh=mesh,
        out_type=jax.ShapeDtypeStruct((2 * N, DIM), jnp.float32),
        scratch_types=[
            pltpu.VMEM((64,), jnp.int32),
            pltpu.VMEM((64,), jnp.int32),
            pltpu.VMEM((64, DIM), jnp.float32),
            pltpu.SemaphoreType.DMA,
        ],
    )(_combine_body)
    return kern(ys, posj)


TBF = 256
TF = N // TBF


def _final_body(x_ref, w_ref, z0_ref, z1_ref, ws1_ref, bs1_ref, ws2_ref,
                bs2_ref, gamma_ref, beta_ref, out_ref):
    x = x_ref[...]
    h = jnp.dot(x, ws1_ref[...], preferred_element_type=jnp.float32) + bs1_ref[...]
    h = _gelu_exact(h)
    sh = jnp.dot(h, ws2_ref[...], preferred_element_type=jnp.float32) + bs2_ref[...]
    w = w_ref[...]
    acc = (x + sh + w[:, 0:1] * z0_ref[0] + w[:, 1:2] * z1_ref[0])
    mu = jnp.mean(acc, axis=1, keepdims=True)
    d = acc - mu
    var = jnp.mean(d * d, axis=1, keepdims=True)
    out_ref[...] = d * jax.lax.rsqrt(var + EPS) * gamma_ref[...] + beta_ref[...]


def _final(x, wts, z3, Ws1, bs1, Ws2, bs2, gamma, beta):
    return pl.pallas_call(
        _final_body,
        grid=(TF,),
        in_specs=[
            pl.BlockSpec((TBF, DIM), lambda t: (t, 0)),
            pl.BlockSpec((TBF, K), lambda t: (t, 0)),
            pl.BlockSpec((1, TBF, DIM), lambda t: (0, t, 0)),
            pl.BlockSpec((1, TBF, DIM), lambda t: (1, t, 0)),
            pl.BlockSpec((DIM, FFN), lambda t: (0, 0)),
            pl.BlockSpec((FFN,), lambda t: (0,)),
            pl.BlockSpec((FFN, DIM), lambda t: (0, 0)),
            pl.BlockSpec((DIM,), lambda t: (0,)),
            pl.BlockSpec((DIM,), lambda t: (0,)),
            pl.BlockSpec((DIM,), lambda t: (0,)),
        ],
        out_specs=pl.BlockSpec((TBF, DIM), lambda t: (t, 0)),
        out_shape=jax.ShapeDtypeStruct((N, DIM), jnp.float32),
    )(x, wts, z3, z3, Ws1, bs1, Ws2, bs2, gamma, beta)


def kernel(x, Wg, W1, b1, W2, b2, Ws1, bs1, Ws2, bs2, gamma, beta):
    ti, wts = _gate(x, Wg)
    tflat = ti.reshape(NK)
    xs, posj, bmeta = _route(tflat, x)
    ys = _experts(bmeta, xs, W1, b1, W2, b2)
    z = _combine(ys, posj)
    z3 = z.reshape(K, N, DIM)
    return _final(x, wts, z3, Ws1, bs1, Ws2, bs2, gamma, beta)


# speedup vs baseline: 1.9104x; 1.1830x over previous
import functools

import jax
import jax.numpy as jnp
from jax import lax
from jax.experimental import pallas as pl
from jax.experimental.pallas import tpu as pltpu
from jax.experimental.pallas import tpu_sc as plsc

DIM = 1024
E = 8
K = 2
FFN = 2048
N = 2048
NK = N * K
EPS = 1e-5

B = 128
NB = 39
NBP = 48
M = NB * B

NC = 2
NS = 16
NW = NC * NS
CH = NK // NS
RW = M // NW
GB = 32
QW = NK // NW

_INV_SQRT2 = 0.7071067811865476


def _gelu_exact(h):
    return 0.5 * h * (1.0 + jax.lax.erf(h * _INV_SQRT2))


def _gate_body(x_ref, wg_ref, ti_ref, w_ref):
    logits = jnp.dot(x_ref[...], wg_ref[...], preferred_element_type=jnp.float32)
    cols = jax.lax.broadcasted_iota(jnp.int32, (N, E), 1)
    m1 = jnp.max(logits, axis=1, keepdims=True)
    i1 = jnp.min(jnp.where(logits == m1, cols, E), axis=1, keepdims=True)
    neg = jnp.float32(-jnp.inf)
    logits2 = jnp.where(cols == i1, neg, logits)
    m2 = jnp.max(logits2, axis=1, keepdims=True)
    i2 = jnp.min(jnp.where(logits2 == m2, cols, E), axis=1, keepdims=True)
    s = jnp.exp(m2 - m1)
    w0 = 1.0 / (1.0 + s)
    w1 = s / (1.0 + s)
    ti_ref[...] = jnp.concatenate([i1, i2], axis=1)
    w_ref[...] = jnp.concatenate([w0, w1], axis=1)


def _gate(x, Wg):
    return pl.pallas_call(
        _gate_body,
        out_shape=(jax.ShapeDtypeStruct((N, K), jnp.int32),
                   jax.ShapeDtypeStruct((N, K), jnp.float32)),
    )(x, Wg)


def _route_body(tflat_hbm, x_hbm, xs_hbm, posj_hbm, bmeta_hbm,
                idsv, tmpv, dest0v, dest1v, histfv, gbuf,
                cnt_s, start_s, csum_s,
                hist_sh, sem):
    cid = lax.axis_index("c")
    sid = lax.axis_index("s")
    wid = sid * NC + cid
    lanes = lax.broadcasted_iota(jnp.int32, (16,), 0)

    pltpu.sync_copy(tflat_hbm.at[pl.ds(sid * CH, CH)], idsv)
    for half in range(2):
        for l in range(16):
            cnt_s[l] = jnp.int32(0)
        for v in range(8):
            vec = idsv[pl.ds(half * 128 + v * 16, 16)]
            for l in range(16):
                e = vec[l]
                cnt_s[e] = cnt_s[e] + 1
        hv = jnp.zeros((16,), jnp.int32)
        for l in range(16):
            hv = jnp.where(lanes == l, cnt_s[l], hv)
        tmpv[...] = hv
        pltpu.sync_copy(tmpv, hist_sh.at[pl.ds(sid * 32 + half * 16, 16)])
    plsc.subcore_barrier()

    pltpu.sync_copy(hist_sh, histfv)
    tot = jnp.zeros((16,), jnp.int32)
    mypre = jnp.zeros((16,), jnp.int32)
    for w in range(NW):
        row = histfv[pl.ds(w * 16, 16)]
        tot = tot + row
        mypre = mypre + jnp.where(w < wid, row, 0)
    padded = ((tot + (B - 1)) >> 7) << 7
    run = jnp.int32(0)
    for e in range(E):
        run = run + padded[e]
        csum_s[e] = run
        start_s[e] = run - padded[e] + mypre[e]

    @pl.when(jnp.logical_and(cid == 0, sid == 0))
    def _bmeta():
        ones = jnp.ones((16,), jnp.int32)
        zeros = jnp.zeros((16,), jnp.int32)
        for half in range(NBP // 16):
            boff = (lanes + half * 16) * B
            acc = jnp.zeros((16,), jnp.int32)
            for e in range(E - 1):
                cev = jnp.where(lanes >= 0, csum_s[e], 0)
                acc = jnp.where(boff >= cev, acc + ones, acc)
            tmpv[...] = jnp.minimum(acc, E - 1)
            pltpu.sync_copy(tmpv, bmeta_hbm.at[0, pl.ds(half * 16, 16)])
            ctv = jnp.where(lanes >= 0, csum_s[E - 1], 0)
            tmpv[...] = jnp.where(boff < ctv, ones, zeros)
            pltpu.sync_copy(tmpv, bmeta_hbm.at[1, pl.ds(half * 16, 16)])

    base = cid * 128
    for tv in range(4):
        dvec0 = jnp.zeros((16,), jnp.int32)
        dvec1 = jnp.zeros((16,), jnp.int32)
        for half in range(2):
            vec = idsv[pl.ds(base + tv * 32 + half * 16, 16)]
            for l in range(16):
                e = vec[l]
                p = start_s[e]
                start_s[e] = p + 1
                lane = half * 8 + (l >> 1)
                if l % 2 == 0:
                    dvec0 = jnp.where(lanes == lane, p, dvec0)
                else:
                    dvec1 = jnp.where(lanes == lane, p, dvec1)
        dest0v[pl.ds(tv * 16, 16)] = dvec0
        dest1v[pl.ds(tv * 16, 16)] = dvec1
    tbase = wid * 64
    pltpu.sync_copy(dest0v, posj_hbm.at[0, pl.ds(tbase, 64)])
    pltpu.sync_copy(dest1v, posj_hbm.at[1, pl.ds(tbase, 64)])

    pltpu.sync_copy(x_hbm.at[pl.ds(tbase, 64)], gbuf)
    pltpu.sync_copy(gbuf, xs_hbm.at[dest0v])
    pltpu.sync_copy(gbuf, xs_hbm.at[dest1v])


def _route(tflat, x):
    mesh = plsc.VectorSubcoreMesh(core_axis_name="c", subcore_axis_name="s")
    kern = functools.partial(
        pl.kernel,
        mesh=mesh,
        out_type=(jax.ShapeDtypeStruct((M, DIM), jnp.float32),
                  jax.ShapeDtypeStruct((2, N), jnp.int32),
                  jax.ShapeDtypeStruct((2, NBP), jnp.int32)),
        scratch_types=[
            pltpu.VMEM((CH,), jnp.int32),
            pltpu.VMEM((16,), jnp.int32),
            pltpu.VMEM((64,), jnp.int32),
            pltpu.VMEM((64,), jnp.int32),
            pltpu.VMEM((NW * 16,), jnp.int32),
            pltpu.VMEM((64, DIM), jnp.float32),
            pltpu.SMEM((16,), jnp.int32),
            pltpu.SMEM((16,), jnp.int32),
            pltpu.SMEM((16,), jnp.int32),
            pltpu.VMEM_SHARED((NW * 16,), jnp.int32),
            pltpu.SemaphoreType.DMA,
        ],
    )(_route_body)
    return kern(tflat, x)


def _expert_body(s_ref, xs_ref, w1_ref, b1_ref, w2_ref, b2_ref, ys_ref):
    b = pl.program_id(0)

    @pl.when(s_ref[1, b] > 0)
    def _():
        h = jnp.dot(xs_ref[...], w1_ref[0], preferred_element_type=jnp.float32)
        h = _gelu_exact(h + b1_ref[0])
        ys_ref[...] = (jnp.dot(h, w2_ref[0], preferred_element_type=jnp.float32)
                       + b2_ref[0])


def _experts(bmeta, xs, W1, b1, W2, b2):
    grid_spec = pltpu.PrefetchScalarGridSpec(
        num_scalar_prefetch=1,
        grid=(NB,),
        in_specs=[
            pl.BlockSpec((B, DIM), lambda b, s: (b, 0)),
            pl.BlockSpec((1, DIM, FFN), lambda b, s: (s[0, b], 0, 0)),
            pl.BlockSpec((1, 1, FFN), lambda b, s: (s[0, b], 0, 0)),
            pl.BlockSpec((1, FFN, DIM), lambda b, s: (s[0, b], 0, 0)),
            pl.BlockSpec((1, 1, DIM), lambda b, s: (s[0, b], 0, 0)),
        ],
        out_specs=pl.BlockSpec((B, DIM), lambda b, s: (b, 0)),
    )
    return pl.pallas_call(
        _expert_body,
        grid_spec=grid_spec,
        out_shape=jax.ShapeDtypeStruct((M, DIM), jnp.float32),
        compiler_params=pltpu.CompilerParams(
            vmem_limit_bytes=100 * 1024 * 1024),
    )(bmeta, xs, W1, b1[:, None, :], W2, b2[:, None, :])


def _combine_body(ys_hbm, posj_hbm, z_hbm, p0v, p1v, gbuf, sem):
    cid = lax.axis_index("c")
    sid = lax.axis_index("s")
    wid = sid * NC + cid
    tbase = wid * 64
    pltpu.sync_copy(posj_hbm.at[0, pl.ds(tbase, 64)], p0v)
    pltpu.sync_copy(posj_hbm.at[1, pl.ds(tbase, 64)], p1v)
    pltpu.async_copy(ys_hbm.at[p0v], gbuf, sem).wait()
    pltpu.sync_copy(gbuf, z_hbm.at[pl.ds(tbase, 64)])
    pltpu.async_copy(ys_hbm.at[p1v], gbuf, sem).wait()
    pltpu.sync_copy(gbuf, z_hbm.at[pl.ds(N + tbase, 64)])


def _combine(ys, posj):
    mesh = plsc.VectorSubcoreMesh(core_axis_name="c", subcore_axis_name="s")
    kern = functools.partial(
        pl.kernel,
        mesh=mesh,
        out_type=jax.ShapeDtypeStruct((2 * N, DIM), jnp.float32),
        scratch_types=[
            pltpu.VMEM((64,), jnp.int32),
            pltpu.VMEM((64,), jnp.int32),
            pltpu.VMEM((64, DIM), jnp.float32),
            pltpu.SemaphoreType.DMA,
        ],
    )(_combine_body)
    return kern(ys, posj)


TBF = 256
TF = N // TBF


def _final_body(x_ref, w_ref, z0_ref, z1_ref, ws1_ref, bs1_ref, ws2_ref,
                bs2_ref, gamma_ref, beta_ref, out_ref):
    x = x_ref[...]
    h = jnp.dot(x, ws1_ref[...], preferred_element_type=jnp.float32) + bs1_ref[...]
    h = _gelu_exact(h)
    sh = jnp.dot(h, ws2_ref[...], preferred_element_type=jnp.float32) + bs2_ref[...]
    w = w_ref[...]
    acc = (x + sh + w[:, 0:1] * z0_ref[0] + w[:, 1:2] * z1_ref[0])
    mu = jnp.mean(acc, axis=1, keepdims=True)
    d = acc - mu
    var = jnp.mean(d * d, axis=1, keepdims=True)
    out_ref[...] = d * jax.lax.rsqrt(var + EPS) * gamma_ref[...] + beta_ref[...]


def _final(x, wts, z3, Ws1, bs1, Ws2, bs2, gamma, beta):
    return pl.pallas_call(
        _final_body,
        grid=(TF,),
        in_specs=[
            pl.BlockSpec((TBF, DIM), lambda t: (t, 0)),
            pl.BlockSpec((TBF, K), lambda t: (t, 0)),
            pl.BlockSpec((1, TBF, DIM), lambda t: (0, t, 0)),
            pl.BlockSpec((1, TBF, DIM), lambda t: (1, t, 0)),
            pl.BlockSpec((DIM, FFN), lambda t: (0, 0)),
            pl.BlockSpec((FFN,), lambda t: (0,)),
            pl.BlockSpec((FFN, DIM), lambda t: (0, 0)),
            pl.BlockSpec((DIM,), lambda t: (0,)),
            pl.BlockSpec((DIM,), lambda t: (0,)),
            pl.BlockSpec((DIM,), lambda t: (0,)),
        ],
        out_specs=pl.BlockSpec((TBF, DIM), lambda t: (t, 0)),
        out_shape=jax.ShapeDtypeStruct((N, DIM), jnp.float32),
    )(x, wts, z3, z3, Ws1, bs1, Ws2, bs2, gamma, beta)


def kernel(x, Wg, W1, b1, W2, b2, Ws1, bs1, Ws2, bs2, gamma, beta):
    ti, wts = _gate(x, Wg)
    tflat = ti.reshape(NK)
    xs, posj, bmeta = _route(tflat, x)
    ys = _experts(bmeta, xs, W1, b1, W2, b2)
    z = _combine(ys, posj)
    z3 = z.reshape(K, N, DIM)
    return _final(x, wts, z3, Ws1, bs1, Ws2, bs2, gamma, beta)


# speedup vs baseline: 1.9658x; 1.0290x over previous
import functools

import jax
import jax.numpy as jnp
from jax import lax
from jax.experimental import pallas as pl
from jax.experimental.pallas import tpu as pltpu
from jax.experimental.pallas import tpu_sc as plsc

DIM = 1024
E = 8
K = 2
FFN = 2048
N = 2048
NK = N * K
EPS = 1e-5

B = 256
NB = 24
NBP = 32
M = NB * B

NC = 2
NS = 16
NW = NC * NS
CH = NK // NS
RW = M // NW
GB = 32
QW = NK // NW

_INV_SQRT2 = 0.7071067811865476


def _gelu_exact(h):
    return 0.5 * h * (1.0 + jax.lax.erf(h * _INV_SQRT2))


def _gate_body(x_ref, wg_ref, ti_ref, w_ref):
    logits = jnp.dot(x_ref[...], wg_ref[...], preferred_element_type=jnp.float32)
    cols = jax.lax.broadcasted_iota(jnp.int32, (N, E), 1)
    m1 = jnp.max(logits, axis=1, keepdims=True)
    i1 = jnp.min(jnp.where(logits == m1, cols, E), axis=1, keepdims=True)
    neg = jnp.float32(-jnp.inf)
    logits2 = jnp.where(cols == i1, neg, logits)
    m2 = jnp.max(logits2, axis=1, keepdims=True)
    i2 = jnp.min(jnp.where(logits2 == m2, cols, E), axis=1, keepdims=True)
    s = jnp.exp(m2 - m1)
    w0 = 1.0 / (1.0 + s)
    w1 = s / (1.0 + s)
    ti_ref[...] = jnp.concatenate([i1, i2], axis=1)
    w_ref[...] = jnp.concatenate([w0, w1], axis=1)


def _gate(x, Wg):
    return pl.pallas_call(
        _gate_body,
        out_shape=(jax.ShapeDtypeStruct((N, K), jnp.int32),
                   jax.ShapeDtypeStruct((N, K), jnp.float32)),
    )(x, Wg)


def _route_body(tflat_hbm, x_hbm, xs_hbm, posj_hbm, bmeta_hbm,
                idsv, tmpv, dest0v, dest1v, histfv, gbuf,
                cnt_s, start_s, csum_s,
                hist_sh, sem):
    cid = lax.axis_index("c")
    sid = lax.axis_index("s")
    wid = sid * NC + cid
    lanes = lax.broadcasted_iota(jnp.int32, (16,), 0)

    pltpu.sync_copy(tflat_hbm.at[pl.ds(sid * CH, CH)], idsv)
    for half in range(2):
        for l in range(16):
            cnt_s[l] = jnp.int32(0)
        for v in range(8):
            vec = idsv[pl.ds(half * 128 + v * 16, 16)]
            for l in range(16):
                e = vec[l]
                cnt_s[e] = cnt_s[e] + 1
        hv = jnp.zeros((16,), jnp.int32)
        for l in range(16):
            hv = jnp.where(lanes == l, cnt_s[l], hv)
        tmpv[...] = hv
        pltpu.sync_copy(tmpv, hist_sh.at[pl.ds(sid * 32 + half * 16, 16)])
    plsc.subcore_barrier()

    pltpu.sync_copy(hist_sh, histfv)
    tot = jnp.zeros((16,), jnp.int32)
    mypre = jnp.zeros((16,), jnp.int32)
    for w in range(NW):
        row = histfv[pl.ds(w * 16, 16)]
        tot = tot + row
        mypre = mypre + jnp.where(w < wid, row, 0)
    padded = ((tot + (B - 1)) >> 8) << 8
    run = jnp.int32(0)
    for e in range(E):
        run = run + padded[e]
        csum_s[e] = run
        start_s[e] = run - padded[e] + mypre[e]

    @pl.when(jnp.logical_and(cid == 0, sid == 0))
    def _bmeta():
        ones = jnp.ones((16,), jnp.int32)
        zeros = jnp.zeros((16,), jnp.int32)
        for half in range(NBP // 16):
            boff = (lanes + half * 16) * B
            acc = jnp.zeros((16,), jnp.int32)
            for e in range(E - 1):
                cev = jnp.where(lanes >= 0, csum_s[e], 0)
                acc = jnp.where(boff >= cev, acc + ones, acc)
            tmpv[...] = jnp.minimum(acc, E - 1)
            pltpu.sync_copy(tmpv, bmeta_hbm.at[0, pl.ds(half * 16, 16)])
            ctv = jnp.where(lanes >= 0, csum_s[E - 1], 0)
            tmpv[...] = jnp.where(boff < ctv, ones, zeros)
            pltpu.sync_copy(tmpv, bmeta_hbm.at[1, pl.ds(half * 16, 16)])

    base = cid * 128
    for tv in range(4):
        dvec0 = jnp.zeros((16,), jnp.int32)
        dvec1 = jnp.zeros((16,), jnp.int32)
        for half in range(2):
            vec = idsv[pl.ds(base + tv * 32 + half * 16, 16)]
            for l in range(16):
                e = vec[l]
                p = start_s[e]
                start_s[e] = p + 1
                lane = half * 8 + (l >> 1)
                if l % 2 == 0:
                    dvec0 = jnp.where(lanes == lane, p, dvec0)
                else:
                    dvec1 = jnp.where(lanes == lane, p, dvec1)
        dest0v[pl.ds(tv * 16, 16)] = dvec0
        dest1v[pl.ds(tv * 16, 16)] = dvec1
    tbase = wid * 64
    pltpu.sync_copy(dest0v, posj_hbm.at[0, pl.ds(tbase, 64)])
    pltpu.sync_copy(dest1v, posj_hbm.at[1, pl.ds(tbase, 64)])

    pltpu.sync_copy(x_hbm.at[pl.ds(tbase, 64)], gbuf)
    pltpu.sync_copy(gbuf, xs_hbm.at[dest0v])
    pltpu.sync_copy(gbuf, xs_hbm.at[dest1v])


def _route(tflat, x):
    mesh = plsc.VectorSubcoreMesh(core_axis_name="c", subcore_axis_name="s")
    kern = functools.partial(
        pl.kernel,
        mesh=mesh,
        out_type=(jax.ShapeDtypeStruct((M, DIM), jnp.float32),
                  jax.ShapeDtypeStruct((2, N), jnp.int32),
                  jax.ShapeDtypeStruct((2, NBP), jnp.int32)),
        scratch_types=[
            pltpu.VMEM((CH,), jnp.int32),
            pltpu.VMEM((16,), jnp.int32),
            pltpu.VMEM((64,), jnp.int32),
            pltpu.VMEM((64,), jnp.int32),
            pltpu.VMEM((NW * 16,), jnp.int32),
            pltpu.VMEM((64, DIM), jnp.float32),
            pltpu.SMEM((16,), jnp.int32),
            pltpu.SMEM((16,), jnp.int32),
            pltpu.SMEM((16,), jnp.int32),
            pltpu.VMEM_SHARED((NW * 16,), jnp.int32),
            pltpu.SemaphoreType.DMA,
        ],
    )(_route_body)
    return kern(tflat, x)


def _expert_body(s_ref, xs_ref, w1_ref, b1_ref, w2_ref, b2_ref, ys_ref):
    b = pl.program_id(0)

    @pl.when(s_ref[1, b] > 0)
    def _():
        h = jnp.dot(xs_ref[...], w1_ref[0], preferred_element_type=jnp.float32)
        h = _gelu_exact(h + b1_ref[0])
        ys_ref[...] = (jnp.dot(h, w2_ref[0], preferred_element_type=jnp.float32)
                       + b2_ref[0])


def _experts(bmeta, xs, W1, b1, W2, b2):
    grid_spec = pltpu.PrefetchScalarGridSpec(
        num_scalar_prefetch=1,
        grid=(NB,),
        in_specs=[
            pl.BlockSpec((B, DIM), lambda b, s: (b, 0)),
            pl.BlockSpec((1, DIM, FFN), lambda b, s: (s[0, b], 0, 0)),
            pl.BlockSpec((1, 1, FFN), lambda b, s: (s[0, b], 0, 0)),
            pl.BlockSpec((1, FFN, DIM), lambda b, s: (s[0, b], 0, 0)),
            pl.BlockSpec((1, 1, DIM), lambda b, s: (s[0, b], 0, 0)),
        ],
        out_specs=pl.BlockSpec((B, DIM), lambda b, s: (b, 0)),
    )
    return pl.pallas_call(
        _expert_body,
        grid_spec=grid_spec,
        out_shape=jax.ShapeDtypeStruct((M, DIM), jnp.float32),
        compiler_params=pltpu.CompilerParams(
            vmem_limit_bytes=100 * 1024 * 1024),
    )(bmeta, xs, W1, b1[:, None, :], W2, b2[:, None, :])


def _combine_body(ys_hbm, posj_hbm, z_hbm, p0v, p1v, gbuf, sem):
    cid = lax.axis_index("c")
    sid = lax.axis_index("s")
    wid = sid * NC + cid
    tbase = wid * 64
    pltpu.sync_copy(posj_hbm.at[0, pl.ds(tbase, 64)], p0v)
    pltpu.sync_copy(posj_hbm.at[1, pl.ds(tbase, 64)], p1v)
    pltpu.async_copy(ys_hbm.at[p0v], gbuf, sem).wait()
    pltpu.sync_copy(gbuf, z_hbm.at[pl.ds(tbase, 64)])
    pltpu.async_copy(ys_hbm.at[p1v], gbuf, sem).wait()
    pltpu.sync_copy(gbuf, z_hbm.at[pl.ds(N + tbase, 64)])


def _combine(ys, posj):
    mesh = plsc.VectorSubcoreMesh(core_axis_name="c", subcore_axis_name="s")
    kern = functools.partial(
        pl.kernel,
        mesh=mesh,
        out_type=jax.ShapeDtypeStruct((2 * N, DIM), jnp.float32),
        scratch_types=[
            pltpu.VMEM((64,), jnp.int32),
            pltpu.VMEM((64,), jnp.int32),
            pltpu.VMEM((64, DIM), jnp.float32),
            pltpu.SemaphoreType.DMA,
        ],
    )(_combine_body)
    return kern(ys, posj)


TBF = 256
TF = N // TBF


def _final_body(x_ref, w_ref, z0_ref, z1_ref, ws1_ref, bs1_ref, ws2_ref,
                bs2_ref, gamma_ref, beta_ref, out_ref):
    x = x_ref[...]
    h = jnp.dot(x, ws1_ref[...], preferred_element_type=jnp.float32) + bs1_ref[...]
    h = _gelu_exact(h)
    sh = jnp.dot(h, ws2_ref[...], preferred_element_type=jnp.float32) + bs2_ref[...]
    w = w_ref[...]
    acc = (x + sh + w[:, 0:1] * z0_ref[0] + w[:, 1:2] * z1_ref[0])
    mu = jnp.mean(acc, axis=1, keepdims=True)
    d = acc - mu
    var = jnp.mean(d * d, axis=1, keepdims=True)
    out_ref[...] = d * jax.lax.rsqrt(var + EPS) * gamma_ref[...] + beta_ref[...]


def _final(x, wts, z3, Ws1, bs1, Ws2, bs2, gamma, beta):
    return pl.pallas_call(
        _final_body,
        grid=(TF,),
        in_specs=[
            pl.BlockSpec((TBF, DIM), lambda t: (t, 0)),
            pl.BlockSpec((TBF, K), lambda t: (t, 0)),
            pl.BlockSpec((1, TBF, DIM), lambda t: (0, t, 0)),
            pl.BlockSpec((1, TBF, DIM), lambda t: (1, t, 0)),
            pl.BlockSpec((DIM, FFN), lambda t: (0, 0)),
            pl.BlockSpec((FFN,), lambda t: (0,)),
            pl.BlockSpec((FFN, DIM), lambda t: (0, 0)),
            pl.BlockSpec((DIM,), lambda t: (0,)),
            pl.BlockSpec((DIM,), lambda t: (0,)),
            pl.BlockSpec((DIM,), lambda t: (0,)),
        ],
        out_specs=pl.BlockSpec((TBF, DIM), lambda t: (t, 0)),
        out_shape=jax.ShapeDtypeStruct((N, DIM), jnp.float32),
    )(x, wts, z3, z3, Ws1, bs1, Ws2, bs2, gamma, beta)


def kernel(x, Wg, W1, b1, W2, b2, Ws1, bs1, Ws2, bs2, gamma, beta):
    ti, wts = _gate(x, Wg)
    tflat = ti.reshape(NK)
    xs, posj, bmeta = _route(tflat, x)
    ys = _experts(bmeta, xs, W1, b1, W2, b2)
    z = _combine(ys, posj)
    z3 = z.reshape(K, N, DIM)
    return _final(x, wts, z3, Ws1, bs1, Ws2, bs2, gamma, beta)
